# Initial kernel scaffold; baseline (speedup 1.0000x reference)
import jax
import jax.numpy as jnp
from jax.experimental import pallas as pl


def kernel(x, pe):
    raise NotImplementedError("write your pallas kernel here")


# speedup vs baseline: 3.2008x; 3.2008x over previous
import jax
import jax.numpy as jnp
from jax.experimental import pallas as pl


def _pe_add_kernel(x_ref, pe_ref, o_ref):
    o_ref[...] = x_ref[...] + pe_ref[...][None, :, :]


def kernel(x, pe):
    batch, seq_len, d_model = x.shape
    s_blk = 1024
    grid = (seq_len // s_blk, batch)
    return pl.pallas_call(
        _pe_add_kernel,
        grid=grid,
        in_specs=[
            pl.BlockSpec((1, s_blk, d_model), lambda s, b: (b, s, 0)),
            pl.BlockSpec((s_blk, d_model), lambda s, b: (s, 0)),
        ],
        out_specs=pl.BlockSpec((1, s_blk, d_model), lambda s, b: (b, s, 0)),
        out_shape=jax.ShapeDtypeStruct(x.shape, x.dtype),
    )(x, pe)


# speedup vs baseline: 3.3149x; 1.0357x over previous
import jax
import jax.numpy as jnp
from jax.experimental import pallas as pl


def _pe_add_kernel(x_ref, pe_ref, o_ref):
    o_ref[...] = x_ref[...] + pe_ref[...][None, :, :]


def kernel(x, pe):
    batch, seq_len, d_model = x.shape
    s_blk = 2048
    grid = (seq_len // s_blk, batch)
    return pl.pallas_call(
        _pe_add_kernel,
        grid=grid,
        in_specs=[
            pl.BlockSpec((1, s_blk, d_model), lambda s, b: (b, s, 0)),
            pl.BlockSpec((s_blk, d_model), lambda s, b: (s, 0)),
        ],
        out_specs=pl.BlockSpec((1, s_blk, d_model), lambda s, b: (b, s, 0)),
        out_shape=jax.ShapeDtypeStruct(x.shape, x.dtype),
    )(x, pe)


# speedup vs baseline: 3.3209x; 1.0018x over previous
import jax
import jax.numpy as jnp
from jax.experimental import pallas as pl


def _pe_add_kernel(x_ref, pe_ref, o_ref):
    o_ref[...] = x_ref[...] + pe_ref[...][None, :, :]


def kernel(x, pe):
    batch, seq_len, d_model = x.shape
    s_blk = 512
    grid = (seq_len // s_blk,)
    return pl.pallas_call(
        _pe_add_kernel,
        grid=grid,
        in_specs=[
            pl.BlockSpec((batch, s_blk, d_model), lambda s: (0, s, 0)),
            pl.BlockSpec((s_blk, d_model), lambda s: (s, 0)),
        ],
        out_specs=pl.BlockSpec((batch, s_blk, d_model), lambda s: (0, s, 0)),
        out_shape=jax.ShapeDtypeStruct(x.shape, x.dtype),
    )(x, pe)
